# baseline (speedup 1.0000x reference)
import functools

import jax
import jax.numpy as jnp
from jax.experimental import pallas as pl
from jax.experimental.pallas import tpu as pltpu

_B, _Q, _C = 64, 900, 256
_QP = 1024
_K = 100
_KP = 128

_NEG = float("-inf")
_BIG = 1 << 30


def _topk_kernel(logits_ref, boxes_ref, ts_ref,
                 scores_ref, labels_ref, oboxes_ref,
                 p_ref, sb_ref):
    x = logits_ref[0]
    m_row = jnp.max(x, axis=1, keepdims=True)
    e = jnp.exp(x - m_row)
    s_row = jnp.sum(e, axis=1, keepdims=True)
    p = e / s_row
    row_id2 = jax.lax.broadcasted_iota(jnp.int32, (_QP, _C), 0)
    p = jnp.where(row_id2 < _Q, p, _NEG)
    p_ref[:] = p

    bx = boxes_ref[0]
    xc, yc = bx[:, 0:1], bx[:, 1:2]
    w2, h2 = bx[:, 2:3] * 0.5, bx[:, 3:4] * 0.5
    xyxy = jnp.concatenate([xc - w2, yc - h2, xc + w2, yc + h2], axis=1)
    img_h = ts_ref[0, 0, 0]
    img_w = ts_ref[0, 0, 1]
    lane4 = jax.lax.broadcasted_iota(jnp.int32, (1, 4), 1)
    scale = jnp.where(lane4 % 2 == 0, img_w, img_h)
    sb_ref[:] = xyxy * scale

    rm0 = jnp.max(p.reshape(8, 128, _C), axis=2)
    flat_rid = jax.lax.broadcasted_iota(jnp.int32, (8, 128), 0) * 128 + \
        jax.lax.broadcasted_iota(jnp.int32, (8, 128), 1)
    col_id = jax.lax.broadcasted_iota(jnp.int32, (1, _C), 1)

    def body(i, rm):
        m = jnp.max(rm)
        r = jnp.min(jnp.where(rm == m, flat_rid, _BIG))
        row = p_ref[pl.ds(r, 1), :]
        c = jnp.min(jnp.where(row == m, col_id, _BIG))
        scores_ref[0, pl.ds(i, 1), :] = m.reshape(1, 1)
        labels_ref[0, pl.ds(i, 1), :] = c.reshape(1, 1)
        oboxes_ref[0, pl.ds(i, 1), :] = sb_ref[pl.ds(r, 1), :]
        masked = jnp.where(col_id == c, _NEG, row)
        p_ref[pl.ds(r, 1), :] = masked
        return jnp.where(flat_rid == r, jnp.max(masked), rm)

    jax.lax.fori_loop(0, _K, body, rm0)


@jax.jit
def kernel(pred_logits, pred_boxes, target_sizes):
    xp = jnp.pad(pred_logits, ((0, 0), (0, _QP - _Q), (0, 0)))
    bp = jnp.pad(pred_boxes, ((0, 0), (0, _QP - _Q), (0, 0)))
    ts = target_sizes.astype(jnp.float32).reshape(_B, 1, 2)

    scores, labels, boxes = pl.pallas_call(
        _topk_kernel,
        grid=(_B,),
        in_specs=[
            pl.BlockSpec((1, _QP, _C), lambda b: (b, 0, 0)),
            pl.BlockSpec((1, _QP, 4), lambda b: (b, 0, 0)),
            pl.BlockSpec((1, 1, 2), lambda b: (b, 0, 0)),
        ],
        out_specs=[
            pl.BlockSpec((1, _KP, 1), lambda b: (b, 0, 0)),
            pl.BlockSpec((1, _KP, 1), lambda b: (b, 0, 0)),
            pl.BlockSpec((1, _KP, 4), lambda b: (b, 0, 0)),
        ],
        out_shape=[
            jax.ShapeDtypeStruct((_B, _KP, 1), jnp.float32),
            jax.ShapeDtypeStruct((_B, _KP, 1), jnp.int32),
            jax.ShapeDtypeStruct((_B, _KP, 4), jnp.float32),
        ],
        scratch_shapes=[
            pltpu.VMEM((_QP, _C), jnp.float32),
            pltpu.VMEM((_QP, 4), jnp.float32),
        ],
        compiler_params=pltpu.CompilerParams(
            dimension_semantics=("parallel",),
        ),
    )(xp, bp, ts)

    return scores[:, :_K, 0], labels[:, :_K, 0], boxes[:, :_K, :]
